# Initial kernel scaffold; baseline (speedup 1.0000x reference)
import jax
import jax.numpy as jnp
from jax.experimental import pallas as pl


def kernel(inputs, W1, b1, W2, b2):
    raise NotImplementedError("write your pallas kernel here")


# speedup vs baseline: 26.9830x; 26.9830x over previous
import functools

import jax
import jax.numpy as jnp
from jax import lax
from jax.experimental import pallas as pl
from jax.experimental.pallas import tpu as pltpu
from jax.experimental.pallas import tpu_sc as plsc

B = 8
N = 2048
FDIM = 64
HID = 32
K = 6
ROWS = 512

NC = 2
NS = 16
NW = NC * NS
QPW = (B * N) // NW
CHUNK = 128
NCHUNK = QPW // CHUNK


def _knn_body(pq_ref, pc_ref, f_ref, w1_ref, b1_ref, knn_ref, g_ref):
    b = pl.program_id(0)
    rt = pl.program_id(1)
    q = pq_ref[0]
    c = pc_ref[0]
    qx = q[:, 0:1]
    qy = q[:, 1:2]
    dx = qx - c[0:1, :]
    dy = qy - c[1:2, :]
    dsq = dx * dx + dy * dy
    colio = lax.broadcasted_iota(jnp.int32, (ROWS, N), 1)
    rowio = lax.broadcasted_iota(jnp.int32, (ROWS, N), 0) + rt * ROWS
    big = jnp.float32(jnp.inf)
    dsq = jnp.where(colio == rowio, big, dsq)
    base = b * N
    cols = []
    for _ in range(K):
        m = jnp.min(dsq, axis=1, keepdims=True)
        cand = jnp.where(dsq == m, colio, N)
        am = jnp.min(cand, axis=1, keepdims=True)
        cols.append(am + base)
        dsq = jnp.where(colio == am, big, dsq)
    knn_ref[0] = jnp.concatenate(cols, axis=1)
    g_ref[0] = (
        jnp.dot(f_ref[0], w1_ref[...], preferred_element_type=jnp.float32)
        + b1_ref[...]
    )


def _knn_call(pq, pc, f, w1, b1):
    return pl.pallas_call(
        _knn_body,
        grid=(B, N // ROWS),
        in_specs=[
            pl.BlockSpec((1, ROWS, 2), lambda b, r: (b, r, 0)),
            pl.BlockSpec((1, 2, N), lambda b, r: (b, 0, 0)),
            pl.BlockSpec((1, ROWS, FDIM), lambda b, r: (b, r, 0)),
            pl.BlockSpec((FDIM, HID), lambda b, r: (0, 0)),
            pl.BlockSpec((1, HID), lambda b, r: (0, 0)),
        ],
        out_specs=[
            pl.BlockSpec((1, ROWS, K), lambda b, r: (b, r, 0)),
            pl.BlockSpec((1, ROWS, HID), lambda b, r: (b, r, 0)),
        ],
        out_shape=[
            jax.ShapeDtypeStruct((B, N, K), jnp.int32),
            jax.ShapeDtypeStruct((B, N, HID), jnp.float32),
        ],
    )(pq, pc, f, w1, b1)


def _make_gather_mean():
    mesh = plsc.VectorSubcoreMesh(core_axis_name="c", subcore_axis_name="s")

    @functools.partial(
        pl.kernel,
        mesh=mesh,
        compiler_params=pltpu.CompilerParams(use_tc_tiling_on_sc=False),
        out_type=jax.ShapeDtypeStruct((B * N, HID), jnp.float32),
        scratch_types=[
            pltpu.VMEM((CHUNK,), jnp.int32),
            pltpu.VMEM((CHUNK, HID), jnp.float32),
            pltpu.VMEM((CHUNK, HID), jnp.float32),
            pltpu.SemaphoreType.DMA,
        ],
    )
    def _gather_mean(idx_hbm, g_hbm, out_hbm, idx_v, rows_v, acc_v, sem):
        wid = lax.axis_index("s") * NC + lax.axis_index("c")
        for chunk in range(NCHUNK):
            base = wid * QPW + chunk * CHUNK
            pltpu.sync_copy(idx_hbm.at[0, pl.ds(base, CHUNK)], idx_v)
            pltpu.async_copy(g_hbm.at[idx_v], acc_v, sem).wait()
            for k in range(1, K):
                pltpu.sync_copy(idx_hbm.at[k, pl.ds(base, CHUNK)], idx_v)
                pltpu.async_copy(g_hbm.at[idx_v], rows_v, sem).wait()

                def _addrow(r, _):
                    for v in range(HID // 16):
                        sl = pl.ds(v * 16, 16)
                        acc_v[r, sl] = acc_v[r, sl] + rows_v[r, sl]
                    return 0

                lax.fori_loop(0, CHUNK, _addrow, 0)
            pltpu.sync_copy(acc_v, out_hbm.at[pl.ds(base, CHUNK)])

    return _gather_mean


def _mlp_body(agg_ref, w2_ref, b2_ref, out_ref):
    h = jnp.maximum(agg_ref[...] * jnp.float32(1.0 / 6.0), 0.0)
    out_ref[...] = (
        jnp.dot(h, w2_ref[...], preferred_element_type=jnp.float32) + b2_ref[...]
    )


def _mlp_call(agg, w2, b2):
    return pl.pallas_call(
        _mlp_body,
        grid=((B * N) // ROWS,),
        in_specs=[
            pl.BlockSpec((ROWS, HID), lambda r: (r, 0)),
            pl.BlockSpec((HID, FDIM), lambda r: (0, 0)),
            pl.BlockSpec((1, FDIM), lambda r: (0, 0)),
        ],
        out_specs=pl.BlockSpec((ROWS, FDIM), lambda r: (r, 0)),
        out_shape=jax.ShapeDtypeStruct((B * N, FDIM), jnp.float32),
    )(agg, w2, b2)


def kernel(inputs, W1, b1, W2, b2):
    pos = inputs[..., :2]
    feats = inputs[..., 2:]
    pc = jnp.transpose(pos, (0, 2, 1))
    knn, g = _knn_call(pos, pc, feats, W1, b1.reshape(1, HID))
    idx_t = jnp.transpose(knn.reshape(B * N, K))
    agg = _make_gather_mean()(idx_t, g.reshape(B * N, HID))
    upd = _mlp_call(agg, W2, b2.reshape(1, FDIM))
    return jnp.concatenate([pos, upd.reshape(B, N, FDIM)], axis=-1)


# speedup vs baseline: 28.6076x; 1.0602x over previous
import functools

import jax
import jax.numpy as jnp
from jax import lax
from jax.experimental import pallas as pl
from jax.experimental.pallas import tpu as pltpu
from jax.experimental.pallas import tpu_sc as plsc

B = 8
N = 2048
FDIM = 64
HID = 32
K = 6
ROWS = 512

NC = 2
NS = 16
NW = NC * NS
QPW = (B * N) // NW
CHUNK = 128
NCHUNK = QPW // CHUNK


def _knn_body(pq_ref, pc_ref, f_ref, w1_ref, b1_ref, knn_ref, g_ref):
    b = pl.program_id(0)
    rt = pl.program_id(1)
    q = pq_ref[0]
    c = pc_ref[0]
    qx = q[:, 0:1]
    qy = q[:, 1:2]
    dx = qx - c[0:1, :]
    dy = qy - c[1:2, :]
    dsq = dx * dx + dy * dy
    colio = lax.broadcasted_iota(jnp.int32, (ROWS, N), 1)
    rowio = lax.broadcasted_iota(jnp.int32, (ROWS, N), 0) + rt * ROWS
    big = jnp.float32(jnp.inf)
    dsq = jnp.where(colio == rowio, big, dsq)
    base = b * N
    cols = []
    for _ in range(K):
        m = jnp.min(dsq, axis=1, keepdims=True)
        mask = dsq == m
        am = jnp.min(jnp.where(mask, colio, N), axis=1, keepdims=True)
        cols.append(am + base)
        dsq = jnp.where(mask, big, dsq)
    knn_ref[0] = jnp.concatenate(cols, axis=1)
    g_ref[0] = (
        jnp.dot(f_ref[0], w1_ref[...], preferred_element_type=jnp.float32)
        + b1_ref[...]
    )


def _knn_call(pq, pc, f, w1, b1):
    return pl.pallas_call(
        _knn_body,
        grid=(B, N // ROWS),
        in_specs=[
            pl.BlockSpec((1, ROWS, 2), lambda b, r: (b, r, 0)),
            pl.BlockSpec((1, 2, N), lambda b, r: (b, 0, 0)),
            pl.BlockSpec((1, ROWS, FDIM), lambda b, r: (b, r, 0)),
            pl.BlockSpec((FDIM, HID), lambda b, r: (0, 0)),
            pl.BlockSpec((1, HID), lambda b, r: (0, 0)),
        ],
        out_specs=[
            pl.BlockSpec((1, ROWS, K), lambda b, r: (b, r, 0)),
            pl.BlockSpec((1, ROWS, HID), lambda b, r: (b, r, 0)),
        ],
        out_shape=[
            jax.ShapeDtypeStruct((B, N, K), jnp.int32),
            jax.ShapeDtypeStruct((B, N, HID), jnp.float32),
        ],
    )(pq, pc, f, w1, b1)


def _make_gather_mean():
    mesh = plsc.VectorSubcoreMesh(core_axis_name="c", subcore_axis_name="s")

    @functools.partial(
        pl.kernel,
        mesh=mesh,
        compiler_params=pltpu.CompilerParams(use_tc_tiling_on_sc=False),
        out_type=jax.ShapeDtypeStruct((B * N, HID), jnp.float32),
        scratch_types=[
            pltpu.VMEM((K * CHUNK,), jnp.int32),
            pltpu.VMEM((K * CHUNK, HID), jnp.float32),
            pltpu.VMEM((CHUNK, HID), jnp.float32),
            pltpu.SemaphoreType.DMA,
        ],
    )
    def _gather_mean(idx_hbm, g_hbm, out_hbm, idx_v, rows_v, acc_v, sem):
        wid = lax.axis_index("s") * NC + lax.axis_index("c")
        for chunk in range(NCHUNK):
            qbase = wid * QPW + chunk * CHUNK
            pltpu.sync_copy(idx_hbm.at[pl.ds(qbase * K, K * CHUNK)], idx_v)
            copies = [
                pltpu.async_copy(
                    g_hbm.at[idx_v.at[pl.ds(c * CHUNK, CHUNK)]],
                    rows_v.at[pl.ds(c * CHUNK, CHUNK)],
                    sem,
                )
                for c in range(K)
            ]
            for cp in copies:
                cp.wait()

            def _addq(q, _):
                r = q * K
                for v in range(HID // 16):
                    sl = pl.ds(v * 16, 16)
                    acc_v[q, sl] = (
                        (rows_v[r, sl] + rows_v[r + 1, sl])
                        + (rows_v[r + 2, sl] + rows_v[r + 3, sl])
                        + (rows_v[r + 4, sl] + rows_v[r + 5, sl])
                    )
                return 0

            lax.fori_loop(0, CHUNK, _addq, 0)
            pltpu.sync_copy(acc_v, out_hbm.at[pl.ds(qbase, CHUNK)])

    return _gather_mean


def _mlp_body(agg_ref, w2_ref, b2_ref, out_ref):
    h = jnp.maximum(agg_ref[...] * jnp.float32(1.0 / 6.0), 0.0)
    out_ref[...] = (
        jnp.dot(h, w2_ref[...], preferred_element_type=jnp.float32) + b2_ref[...]
    )


def _mlp_call(agg, w2, b2):
    return pl.pallas_call(
        _mlp_body,
        grid=((B * N) // ROWS,),
        in_specs=[
            pl.BlockSpec((ROWS, HID), lambda r: (r, 0)),
            pl.BlockSpec((HID, FDIM), lambda r: (0, 0)),
            pl.BlockSpec((1, FDIM), lambda r: (0, 0)),
        ],
        out_specs=pl.BlockSpec((ROWS, FDIM), lambda r: (r, 0)),
        out_shape=jax.ShapeDtypeStruct((B * N, FDIM), jnp.float32),
    )(agg, w2, b2)


def kernel(inputs, W1, b1, W2, b2):
    pos = inputs[..., :2]
    feats = inputs[..., 2:]
    pc = jnp.transpose(pos, (0, 2, 1))
    knn, g = _knn_call(pos, pc, feats, W1, b1.reshape(1, HID))
    agg = _make_gather_mean()(knn.reshape(B * N * K), g.reshape(B * N, HID))
    upd = _mlp_call(agg, W2, b2.reshape(1, FDIM))
    return jnp.concatenate([pos, upd.reshape(B, N, FDIM)], axis=-1)


# speedup vs baseline: 31.6172x; 1.1052x over previous
import functools

import jax
import jax.numpy as jnp
from jax import lax
from jax.experimental import pallas as pl
from jax.experimental.pallas import tpu as pltpu
from jax.experimental.pallas import tpu_sc as plsc

B = 8
N = 2048
FDIM = 64
HID = 32
K = 6
ROWS = 512

NC = 2
NS = 16
NW = NC * NS
QPW = (B * N) // NW
CHUNK = 128
NCHUNK = QPW // CHUNK


def _knn_body(pq_ref, pc_ref, f_ref, w1_ref, b1_ref, knn_ref, g_ref):
    b = pl.program_id(0)
    rt = pl.program_id(1)
    q = pq_ref[0]
    c = pc_ref[0]
    qx = q[:, 0:1]
    qy = q[:, 1:2]
    dx = qx - c[0:1, :]
    dy = qy - c[1:2, :]
    dsq = dx * dx + dy * dy
    colio = lax.broadcasted_iota(jnp.int32, (ROWS, N), 1)
    rowio = lax.broadcasted_iota(jnp.int32, (ROWS, N), 0) + rt * ROWS
    colf = colio.astype(jnp.float32)
    big = jnp.float32(jnp.inf)
    bigf = jnp.float32(float(N))
    dsq = jnp.where(colio == rowio, big, dsq)
    base = b * N
    cols = []
    for _ in range(K):
        m = jnp.min(dsq, axis=1, keepdims=True)
        cand = jnp.where(dsq == m, colf, bigf)
        am = jnp.min(cand, axis=1, keepdims=True)
        cols.append(am)
        dsq = jnp.where(cand == am, big, dsq)
    knn_ref[0] = jnp.concatenate(cols, axis=1).astype(jnp.int32) + base
    g_ref[0] = (
        jnp.dot(f_ref[0], w1_ref[...], preferred_element_type=jnp.float32)
        + b1_ref[...]
    )


def _knn_call(pq, pc, f, w1, b1):
    return pl.pallas_call(
        _knn_body,
        grid=(B, N // ROWS),
        in_specs=[
            pl.BlockSpec((1, ROWS, 2), lambda b, r: (b, r, 0)),
            pl.BlockSpec((1, 2, N), lambda b, r: (b, 0, 0)),
            pl.BlockSpec((1, ROWS, FDIM), lambda b, r: (b, r, 0)),
            pl.BlockSpec((FDIM, HID), lambda b, r: (0, 0)),
            pl.BlockSpec((1, HID), lambda b, r: (0, 0)),
        ],
        out_specs=[
            pl.BlockSpec((1, ROWS, K), lambda b, r: (b, r, 0)),
            pl.BlockSpec((1, ROWS, HID), lambda b, r: (b, r, 0)),
        ],
        out_shape=[
            jax.ShapeDtypeStruct((B, N, K), jnp.int32),
            jax.ShapeDtypeStruct((B, N, HID), jnp.float32),
        ],
    )(pq, pc, f, w1, b1)


def _make_gather_mean():
    mesh = plsc.VectorSubcoreMesh(core_axis_name="c", subcore_axis_name="s")

    @functools.partial(
        pl.kernel,
        mesh=mesh,
        compiler_params=pltpu.CompilerParams(use_tc_tiling_on_sc=False),
        out_type=jax.ShapeDtypeStruct((B * N, HID), jnp.float32),
        scratch_types=[
            pltpu.VMEM((K * CHUNK,), jnp.int32),
            pltpu.VMEM((K * CHUNK, HID), jnp.float32),
            pltpu.VMEM((CHUNK, HID), jnp.float32),
            pltpu.SemaphoreType.DMA,
        ],
    )
    def _gather_mean(idx_hbm, g_hbm, out_hbm, idx_v, rows_v, acc_v, sem):
        wid = lax.axis_index("s") * NC + lax.axis_index("c")
        for chunk in range(NCHUNK):
            qbase = wid * QPW + chunk * CHUNK
            pltpu.sync_copy(idx_hbm.at[pl.ds(qbase * K, K * CHUNK)], idx_v)
            copies = [
                pltpu.async_copy(
                    g_hbm.at[idx_v.at[pl.ds(c * CHUNK, CHUNK)]],
                    rows_v.at[pl.ds(c * CHUNK, CHUNK)],
                    sem,
                )
                for c in range(K)
            ]
            for cp in copies:
                cp.wait()

            def _addq(q, _):
                r = q * K
                for v in range(HID // 16):
                    sl = pl.ds(v * 16, 16)
                    acc_v[q, sl] = (
                        (rows_v[r, sl] + rows_v[r + 1, sl])
                        + (rows_v[r + 2, sl] + rows_v[r + 3, sl])
                        + (rows_v[r + 4, sl] + rows_v[r + 5, sl])
                    )
                return 0

            lax.fori_loop(0, CHUNK, _addq, 0)
            pltpu.sync_copy(acc_v, out_hbm.at[pl.ds(qbase, CHUNK)])

    return _gather_mean


def _mlp_body(agg_ref, w2_ref, b2_ref, out_ref):
    h = jnp.maximum(agg_ref[...] * jnp.float32(1.0 / 6.0), 0.0)
    out_ref[...] = (
        jnp.dot(h, w2_ref[...], preferred_element_type=jnp.float32) + b2_ref[...]
    )


def _mlp_call(agg, w2, b2):
    return pl.pallas_call(
        _mlp_body,
        grid=((B * N) // ROWS,),
        in_specs=[
            pl.BlockSpec((ROWS, HID), lambda r: (r, 0)),
            pl.BlockSpec((HID, FDIM), lambda r: (0, 0)),
            pl.BlockSpec((1, FDIM), lambda r: (0, 0)),
        ],
        out_specs=pl.BlockSpec((ROWS, FDIM), lambda r: (r, 0)),
        out_shape=jax.ShapeDtypeStruct((B * N, FDIM), jnp.float32),
    )(agg, w2, b2)


def kernel(inputs, W1, b1, W2, b2):
    pos = inputs[..., :2]
    feats = inputs[..., 2:]
    pc = jnp.transpose(pos, (0, 2, 1))
    knn, g = _knn_call(pos, pc, feats, W1, b1.reshape(1, HID))
    agg = _make_gather_mean()(knn.reshape(B * N * K), g.reshape(B * N, HID))
    upd = _mlp_call(agg, W2, b2.reshape(1, FDIM))
    return jnp.concatenate([pos, upd.reshape(B, N, FDIM)], axis=-1)
